# Initial kernel scaffold; baseline (speedup 1.0000x reference)
import jax
import jax.numpy as jnp
from jax.experimental import pallas as pl


def kernel(x):
    raise NotImplementedError("write your pallas kernel here")


# speedup vs baseline: 9.4853x; 9.4853x over previous
import functools

import jax
import jax.numpy as jnp
from jax.experimental import pallas as pl

_K_RATIO = 0.1


def _topk_mask_norm_body(x_ref, o_ref, *, k):
    x = x_ref[...]
    bits = jax.lax.bitcast_convert_type(x, jnp.int32) & jnp.int32(0x7FFFFFFF)

    r = x.shape[0]

    def body(_, carry):
        lo, hi = carry
        mid = lo + ((hi - lo + 1) >> 1)
        cnt = jnp.sum((bits >= mid).astype(jnp.int32), axis=-1, keepdims=True)
        ge = cnt >= k
        lo = jnp.where(ge, mid, lo)
        hi = jnp.where(ge, hi, mid - 1)
        return lo, hi

    lo0 = jnp.zeros((r, 1), jnp.int32)
    hi0 = jnp.full((r, 1), 0x7F800000, jnp.int32)
    lo, _ = jax.lax.fori_loop(0, 31, body, (lo0, hi0))

    mask = bits >= lo
    xm = jnp.where(mask, x, 0.0)
    ss = jnp.sum(xm * xm, axis=-1, keepdims=True)
    o_ref[...] = xm / (jnp.sqrt(ss) + 1e-6)


@jax.jit
def kernel(x):
    m, n = x.shape
    k = int(_K_RATIO * n)
    r = 8
    return pl.pallas_call(
        functools.partial(_topk_mask_norm_body, k=k),
        grid=(m // r,),
        in_specs=[pl.BlockSpec((r, n), lambda i: (i, 0))],
        out_specs=pl.BlockSpec((r, n), lambda i: (i, 0)),
        out_shape=jax.ShapeDtypeStruct((m, n), jnp.float32),
    )(x)
